# Initial kernel scaffold; baseline (speedup 1.0000x reference)
import jax
import jax.numpy as jnp
from jax.experimental import pallas as pl


def kernel(x, table, W1, b1, W2, b2):
    raise NotImplementedError("write your pallas kernel here")


# speedup vs baseline: 147.0771x; 147.0771x over previous
import jax
import jax.numpy as jnp
from jax.experimental import pallas as pl
from jax.experimental.pallas import tpu as pltpu


_NUM_VALS = 21


def _body(x_ref, t_ref, w1_ref, b1_ref, w2_ref, b2_ref, out_ref, a_ref,
          cnt_ref):
    i = pl.program_id(0)

    @pl.when(i == 0)
    def _init():
        cnt_ref[...] = jnp.zeros_like(cnt_ref)

    xb = x_ref[...]
    for v in range(1, _NUM_VALS):
        cnt_ref[v, :] += jnp.sum((xb == v).astype(jnp.float32), axis=0)

    @pl.when(i == pl.num_programs(0) - 1)
    def _finish():
        counts = cnt_ref[...]
        a = jax.lax.dot_general(
            counts, t_ref[...], (((0,), (0,)), ((), ())),
            preferred_element_type=jnp.float32)
        a_ref[...] = a
        h = jnp.tanh(
            jax.lax.dot(a, w1_ref[...], preferred_element_type=jnp.float32)
            + b1_ref[...])
        out_ref[...] = (
            jax.lax.dot(h, w2_ref[...], preferred_element_type=jnp.float32)
            + b2_ref[...])


def kernel(x, table, W1, b1, W2, b2):
    B, L = x.shape
    V, D = table.shape
    H = W1.shape[1]
    BB = 512
    grid = (B // BB,)

    out, a = pl.pallas_call(
        _body,
        grid=grid,
        in_specs=[
            pl.BlockSpec((BB, L), lambda i: (i, 0)),
            pl.BlockSpec((V, D), lambda i: (0, 0)),
            pl.BlockSpec((D, H), lambda i: (0, 0)),
            pl.BlockSpec((1, H), lambda i: (0, 0)),
            pl.BlockSpec((H, D), lambda i: (0, 0)),
            pl.BlockSpec((1, D), lambda i: (0, 0)),
        ],
        out_specs=[
            pl.BlockSpec((L, D), lambda i: (0, 0)),
            pl.BlockSpec((L, D), lambda i: (0, 0)),
        ],
        out_shape=[
            jax.ShapeDtypeStruct((L, D), jnp.float32),
            jax.ShapeDtypeStruct((L, D), jnp.float32),
        ],
        scratch_shapes=[pltpu.VMEM((V, L), jnp.float32)],
    )(x, table, W1, b1.reshape(1, H), W2, b2.reshape(1, D))
    return (out, a)
